# baseline (speedup 1.0000x reference)
import functools

import jax
import jax.numpy as jnp
from jax import lax
from jax.experimental import pallas as pl
from jax.experimental.pallas import tpu as pltpu
from jax.experimental.pallas import tpu_sc as plsc

_HIDDEN = 64
_BCHUNK = 8
_TC_COLS = 32768


@functools.cache
def _make_transpose(vocab: int):
    grid = (vocab + _TC_COLS - 1) // _TC_COLS

    def body(t_ref, o_ref):
        rows = t_ref[...].T
        o_ref[...] = jnp.concatenate([rows, jnp.zeros_like(rows)], axis=1)

    return pl.pallas_call(
        body,
        grid=(grid,),
        in_specs=[pl.BlockSpec((_HIDDEN, _TC_COLS), lambda i: (0, i))],
        out_specs=pl.BlockSpec((_TC_COLS, 2 * _HIDDEN), lambda i: (i, 0)),
        out_shape=jax.ShapeDtypeStruct((vocab, 2 * _HIDDEN), jnp.float32),
    )


@functools.cache
def _make_gather(bsz: int, seq: int, vocab: int):
    info = plsc.get_sparse_core_info()
    nc, ns = info.num_cores, info.num_subcores
    nw = nc * ns
    b_per_w = bsz // nw
    n_chunks = b_per_w // _BCHUNK
    assert b_per_w * nw == bsz and n_chunks * _BCHUNK == b_per_w
    assert n_chunks % 2 == 0 and n_chunks >= 4

    mesh = plsc.VectorSubcoreMesh(core_axis_name="c", subcore_axis_name="s")

    @functools.partial(
        pl.kernel,
        out_type=jax.ShapeDtypeStruct((bsz * seq, 2 * _HIDDEN), jnp.float32),
        mesh=mesh,
        compiler_params=pltpu.CompilerParams(use_tc_tiling_on_sc=True),
        scratch_types=[
            pltpu.VMEM((b_per_w, seq), jnp.int32),
            pltpu.VMEM((_BCHUNK * seq, 2 * _HIDDEN), jnp.float32),
            pltpu.VMEM((_BCHUNK * seq, 2 * _HIDDEN), jnp.float32),
            pltpu.SemaphoreType.DMA,
            pltpu.SemaphoreType.DMA,
            pltpu.SemaphoreType.DMA,
        ],
    )
    def gather_kernel(idx_hbm, table_hbm, out_hbm, idx_v, b0, b1, g0, g1, ssem):
        wid = lax.axis_index("s") * nc + lax.axis_index("c")
        base = wid * b_per_w
        pltpu.sync_copy(idx_hbm.at[pl.ds(base, b_per_w)], idx_v)

        bufs = (b0, b1)
        gsems = (g0, g1)

        def fire_chunk(c, q):
            for i in range(_BCHUNK):
                pltpu.async_copy(
                    table_hbm.at[idx_v.at[c * _BCHUNK + i]],
                    bufs[q].at[pl.ds(i * seq, seq)],
                    gsems[q],
                )

        def drain_chunk(q):
            for i in range(_BCHUNK):
                pltpu.make_async_copy(
                    table_hbm.at[idx_v.at[0]], bufs[q].at[pl.ds(0, seq)], gsems[q]
                ).wait()

        def fire_scatter(c, q):
            pltpu.async_copy(
                bufs[q],
                out_hbm.at[pl.ds((base + c * _BCHUNK) * seq, _BCHUNK * seq)],
                ssem,
            )

        def drain_scatter(q):
            pltpu.make_async_copy(
                bufs[q], out_hbm.at[pl.ds(0, _BCHUNK * seq)], ssem
            ).wait()

        fire_chunk(0, 0)
        fire_chunk(1, 1)
        drain_chunk(0)
        fire_scatter(0, 0)

        @pl.loop(0, n_chunks // 2 - 1)
        def _steady(t):
            p = 2 * t + 1
            drain_scatter(0)
            fire_chunk(p + 1, 0)
            drain_chunk(1)
            fire_scatter(p, 1)
            drain_scatter(1)
            fire_chunk(p + 2, 1)
            drain_chunk(0)
            fire_scatter(p + 1, 0)

        drain_scatter(0)
        drain_chunk(1)
        fire_scatter(n_chunks - 1, 1)
        drain_scatter(1)

    return gather_kernel


def kernel(batch, positions, mask, table):
    bsz, seq = batch.shape
    vocab = table.shape[0]
    staged = _make_transpose(vocab)(table.T)
    wide = _make_gather(bsz, seq, vocab)(batch, staged)
    return wide.reshape(bsz, seq, 2 * _HIDDEN)[:, :, :_HIDDEN]


# speedup vs baseline: 1.2677x; 1.2677x over previous
import functools

import jax
import jax.numpy as jnp
from jax import lax
from jax.experimental import pallas as pl
from jax.experimental.pallas import tpu as pltpu
from jax.experimental.pallas import tpu_sc as plsc

_HIDDEN = 64
_BCHUNK = 4
_TC_COLS = 32768


@functools.cache
def _make_transpose(vocab: int):
    grid = (vocab + _TC_COLS - 1) // _TC_COLS

    def body(t_ref, o_ref):
        rows = t_ref[...].T
        o_ref[...] = jnp.concatenate([rows, jnp.zeros_like(rows)], axis=1)

    return pl.pallas_call(
        body,
        grid=(grid,),
        in_specs=[pl.BlockSpec((_HIDDEN, _TC_COLS), lambda i: (0, i))],
        out_specs=pl.BlockSpec((_TC_COLS, 2 * _HIDDEN), lambda i: (i, 0)),
        out_shape=jax.ShapeDtypeStruct((vocab, 2 * _HIDDEN), jnp.float32),
    )


@functools.cache
def _make_gather(bsz: int, seq: int, vocab: int):
    info = plsc.get_sparse_core_info()
    nc, ns = info.num_cores, info.num_subcores
    nw = nc * ns
    b_per_w = bsz // nw
    n_chunks = b_per_w // _BCHUNK
    assert b_per_w * nw == bsz and n_chunks * _BCHUNK == b_per_w
    assert n_chunks % 2 == 0 and n_chunks >= 4

    mesh = plsc.VectorSubcoreMesh(core_axis_name="c", subcore_axis_name="s")

    @functools.partial(
        pl.kernel,
        out_type=jax.ShapeDtypeStruct((bsz, seq, 2 * _HIDDEN), jnp.float32),
        mesh=mesh,
        compiler_params=pltpu.CompilerParams(use_tc_tiling_on_sc=True),
        scratch_types=[
            pltpu.VMEM((b_per_w, seq), jnp.int32),
            pltpu.VMEM((_BCHUNK, seq, 2 * _HIDDEN), jnp.float32),
            pltpu.VMEM((_BCHUNK, seq, 2 * _HIDDEN), jnp.float32),
            pltpu.SemaphoreType.DMA,
            pltpu.SemaphoreType.DMA,
            pltpu.SemaphoreType.DMA,
        ],
    )
    def gather_kernel(idx_hbm, table_hbm, out_hbm, idx_v, b0, b1, g0, g1, ssem):
        wid = lax.axis_index("s") * nc + lax.axis_index("c")
        base = wid * b_per_w
        pltpu.sync_copy(idx_hbm.at[pl.ds(base, b_per_w)], idx_v)

        bufs = (b0, b1)
        gsems = (g0, g1)

        def fire_chunk(c, q):
            for i in range(_BCHUNK):
                pltpu.async_copy(
                    table_hbm.at[idx_v.at[c * _BCHUNK + i]],
                    bufs[q].at[i],
                    gsems[q],
                )

        def drain_chunk(q):
            for i in range(_BCHUNK):
                pltpu.make_async_copy(
                    table_hbm.at[idx_v.at[0]], bufs[q].at[i], gsems[q]
                ).wait()

        def fire_scatter(c, q):
            pltpu.async_copy(
                bufs[q],
                out_hbm.at[pl.ds(base + c * _BCHUNK, _BCHUNK)],
                ssem,
            )

        def drain_scatter(q):
            pltpu.make_async_copy(
                bufs[q], out_hbm.at[pl.ds(0, _BCHUNK)], ssem
            ).wait()

        fire_chunk(0, 0)
        fire_chunk(1, 1)
        drain_chunk(0)
        fire_scatter(0, 0)

        @pl.loop(0, n_chunks // 2 - 1)
        def _steady(t):
            p = 2 * t + 1
            drain_scatter(0)
            fire_chunk(p + 1, 0)
            drain_chunk(1)
            fire_scatter(p, 1)
            drain_scatter(1)
            fire_chunk(p + 2, 1)
            drain_chunk(0)
            fire_scatter(p + 1, 0)

        drain_scatter(0)
        drain_chunk(1)
        fire_scatter(n_chunks - 1, 1)
        drain_scatter(1)

    return gather_kernel


def kernel(batch, positions, mask, table):
    bsz, seq = batch.shape
    vocab = table.shape[0]
    staged = _make_transpose(vocab)(table.T)
    wide = _make_gather(bsz, seq, vocab)(batch, staged)
    return wide[:, :, :_HIDDEN]


# speedup vs baseline: 1.2749x; 1.0057x over previous
import functools

import jax
import jax.numpy as jnp
from jax import lax
from jax.experimental import pallas as pl
from jax.experimental.pallas import tpu as pltpu
from jax.experimental.pallas import tpu_sc as plsc

_HIDDEN = 64
_BCHUNK = 8
_TC_COLS = 32768


@functools.cache
def _make_transpose(vocab: int):
    grid = (vocab + _TC_COLS - 1) // _TC_COLS

    def body(t_ref, o_ref):
        rows = t_ref[...].T
        o_ref[...] = jnp.concatenate([rows, jnp.zeros_like(rows)], axis=1)

    return pl.pallas_call(
        body,
        grid=(grid,),
        in_specs=[pl.BlockSpec((_HIDDEN, _TC_COLS), lambda i: (0, i))],
        out_specs=pl.BlockSpec((_TC_COLS, 2 * _HIDDEN), lambda i: (i, 0)),
        out_shape=jax.ShapeDtypeStruct((vocab, 2 * _HIDDEN), jnp.float32),
    )


@functools.cache
def _make_gather(bsz: int, seq: int, vocab: int):
    info = plsc.get_sparse_core_info()
    nc, ns = info.num_cores, info.num_subcores
    nw = nc * ns
    b_per_w = bsz // nw
    n_chunks = b_per_w // _BCHUNK
    assert b_per_w * nw == bsz and n_chunks * _BCHUNK == b_per_w
    assert n_chunks % 2 == 0 and n_chunks >= 4

    mesh = plsc.VectorSubcoreMesh(core_axis_name="c", subcore_axis_name="s")

    @functools.partial(
        pl.kernel,
        out_type=jax.ShapeDtypeStruct((bsz, seq, 2 * _HIDDEN), jnp.float32),
        mesh=mesh,
        compiler_params=pltpu.CompilerParams(use_tc_tiling_on_sc=True),
        scratch_types=[
            pltpu.VMEM((b_per_w, seq), jnp.int32),
            pltpu.VMEM((_BCHUNK, seq, 2 * _HIDDEN), jnp.float32),
            pltpu.VMEM((_BCHUNK, seq, 2 * _HIDDEN), jnp.float32),
            pltpu.SemaphoreType.DMA,
            pltpu.SemaphoreType.DMA,
            pltpu.SemaphoreType.DMA,
        ],
    )
    def gather_kernel(idx_hbm, table_hbm, out_hbm, idx_v, b0, b1, g0, g1, ssem):
        wid = lax.axis_index("s") * nc + lax.axis_index("c")
        base = wid * b_per_w
        pltpu.sync_copy(idx_hbm.at[pl.ds(base, b_per_w)], idx_v)

        bufs = (b0, b1)
        gsems = (g0, g1)

        def fire_chunk(c, q):
            for i in range(_BCHUNK):
                pltpu.async_copy(
                    table_hbm.at[idx_v.at[c * _BCHUNK + i]],
                    bufs[q].at[i],
                    gsems[q],
                )

        def drain_chunk(q):
            for i in range(_BCHUNK):
                pltpu.make_async_copy(
                    table_hbm.at[idx_v.at[0]], bufs[q].at[i], gsems[q]
                ).wait()

        def fire_scatter(c, q):
            pltpu.async_copy(
                bufs[q],
                out_hbm.at[pl.ds(base + c * _BCHUNK, _BCHUNK)],
                ssem,
            )

        def drain_scatter(q):
            pltpu.make_async_copy(
                bufs[q], out_hbm.at[pl.ds(0, _BCHUNK)], ssem
            ).wait()

        fire_chunk(0, 0)
        fire_chunk(1, 1)
        drain_chunk(0)
        fire_scatter(0, 0)

        @pl.loop(0, n_chunks // 2 - 1)
        def _steady(t):
            p = 2 * t + 1
            drain_scatter(0)
            fire_chunk(p + 1, 0)
            drain_chunk(1)
            fire_scatter(p, 1)
            drain_scatter(1)
            fire_chunk(p + 2, 1)
            drain_chunk(0)
            fire_scatter(p + 1, 0)

        drain_scatter(0)
        drain_chunk(1)
        fire_scatter(n_chunks - 1, 1)
        drain_scatter(1)

    return gather_kernel


def kernel(batch, positions, mask, table):
    bsz, seq = batch.shape
    vocab = table.shape[0]
    staged = _make_transpose(vocab)(table.T)
    wide = _make_gather(bsz, seq, vocab)(batch, staged)
    return wide[:, :, :_HIDDEN]
